# baseline (speedup 1.0000x reference)
import functools

import jax
import jax.numpy as jnp
from jax.experimental import pallas as pl
from jax.experimental.pallas import tpu as pltpu

_BM = 256
_BK = 1024


def _augment_kernel(c_ref, c2_ref):
    c = c_ref[:]
    d = c.shape[1]
    c2_ref[:, 0:d] = c
    c2_ref[:, d:d + 1] = jnp.sum(c * c, axis=1, keepdims=True)


def _vq_loss_kernel(zt_ref, c2_ref, out_ref, *, n_tokens, k_codes):
    ztb = zt_ref[:]
    z2 = jnp.concatenate(
        [-2.0 * ztb, jnp.ones((1, ztb.shape[1]), jnp.float32)], axis=0)

    def body(i, minv):
        c2 = c2_ref[pl.ds(i * _BK, _BK), :]
        part = jnp.dot(c2, z2, preferred_element_type=jnp.float32)
        return jnp.minimum(minv, jnp.min(part, axis=0, keepdims=True))

    minv = jax.lax.fori_loop(
        0, k_codes // _BK, body,
        jnp.full((1, ztb.shape[1]), jnp.inf, dtype=jnp.float32))
    zsq = jnp.sum(ztb * ztb, axis=0, keepdims=True)
    s = jnp.sum(minv + zsq) / n_tokens
    out_ref[:] = jnp.full(out_ref.shape, s, dtype=jnp.float32)


def kernel(z, codebook):
    n, d = z.shape
    k = codebook.shape[0]
    zt = z.T
    c2 = pl.pallas_call(
        _augment_kernel,
        out_shape=jax.ShapeDtypeStruct((k, d + 1), jnp.float32),
    )(codebook)
    gm = n // _BM
    parts = pl.pallas_call(
        functools.partial(_vq_loss_kernel, n_tokens=n, k_codes=k),
        grid=(gm,),
        in_specs=[
            pl.BlockSpec((d, _BM), lambda m: (0, m)),
            pl.BlockSpec((k, d + 1), lambda m: (0, 0)),
        ],
        out_specs=pl.BlockSpec((1, 8, 128), lambda m: (m, 0, 0)),
        out_shape=jax.ShapeDtypeStruct((gm, 8, 128), jnp.float32),
        compiler_params=pltpu.CompilerParams(
            dimension_semantics=("parallel",)),
    )(zt, c2)
    return jnp.sum(parts[:, 0, 0])


# speedup vs baseline: 1.0112x; 1.0112x over previous
import functools

import jax
import jax.numpy as jnp
from jax.experimental import pallas as pl
from jax.experimental.pallas import tpu as pltpu

_BM = 256
_BK = 1024


def _vq_loss_kernel(zt_ref, c_ref, out_ref, c2_ref, *, n_tokens, k_codes):
    d = zt_ref.shape[0]

    @pl.when(pl.program_id(0) == 0)
    def _augment():
        c = c_ref[:]
        c2_ref[:, 0:d] = c.astype(jnp.bfloat16)
        c2_ref[:, d:d + 1] = jnp.sum(c * c, axis=1, keepdims=True
                                     ).astype(jnp.bfloat16)

    ztb = zt_ref[:]
    z2 = jnp.concatenate(
        [-2.0 * ztb, jnp.ones((1, ztb.shape[1]), jnp.float32)],
        axis=0).astype(jnp.bfloat16)

    def body(i, minv):
        c2 = c2_ref[pl.ds(i * _BK, _BK), :]
        part = jnp.dot(c2, z2, preferred_element_type=jnp.float32)
        return jnp.minimum(minv, jnp.min(part, axis=0, keepdims=True))

    minv = jax.lax.fori_loop(
        0, k_codes // _BK, body,
        jnp.full((1, ztb.shape[1]), jnp.inf, dtype=jnp.float32))
    zsq = jnp.sum(ztb * ztb, axis=0, keepdims=True)
    s = jnp.sum(minv + zsq)

    @pl.when(pl.program_id(0) == 0)
    def _init():
        out_ref[0, 0] = 0.0

    out_ref[0, 0] += s / n_tokens


def kernel(z, codebook):
    n, d = z.shape
    k = codebook.shape[0]
    zt = z.T
    out = pl.pallas_call(
        functools.partial(_vq_loss_kernel, n_tokens=n, k_codes=k),
        grid=(n // _BM,),
        in_specs=[
            pl.BlockSpec((d, _BM), lambda m: (0, m)),
            pl.BlockSpec((k, d), lambda m: (0, 0)),
        ],
        out_specs=pl.BlockSpec(memory_space=pltpu.SMEM),
        out_shape=jax.ShapeDtypeStruct((1, 1), jnp.float32),
        scratch_shapes=[pltpu.VMEM((k, d + 1), jnp.bfloat16)],
    )(zt, codebook)
    return out[0, 0]


# speedup vs baseline: 1.7041x; 1.6852x over previous
import functools

import jax
import jax.numpy as jnp
from jax.experimental import pallas as pl
from jax.experimental.pallas import tpu as pltpu

_BM = 256
_BK = 1024


def _vq_loss_kernel(zt_ref, c_ref, out_ref, c2_ref, *, n_tokens, k_codes):
    d = zt_ref.shape[0]

    @pl.when(pl.program_id(0) == 0)
    def _augment():
        c = c_ref[:]
        c2_ref[:, 0:d] = c.astype(jnp.bfloat16)
        c2_ref[:, d:d + 1] = jnp.sum(c * c, axis=1, keepdims=True
                                     ).astype(jnp.bfloat16)

    ztb = zt_ref[:]
    z2 = jnp.concatenate(
        [-2.0 * ztb, jnp.ones((1, ztb.shape[1]), jnp.float32)],
        axis=0).astype(jnp.bfloat16)

    def body(i, minv):
        c2 = c2_ref[pl.ds(i * _BK, _BK), :]
        part = jnp.dot(c2, z2, preferred_element_type=jnp.float32)
        return jnp.minimum(minv, jnp.min(part, axis=0, keepdims=True))

    minv = jax.lax.fori_loop(
        0, k_codes // _BK, body,
        jnp.full((1, ztb.shape[1]), jnp.inf, dtype=jnp.float32),
        unroll=4)
    zsq = jnp.sum(ztb * ztb, axis=0, keepdims=True)
    s = jnp.sum(minv + zsq)

    @pl.when(pl.program_id(0) == 0)
    def _init():
        out_ref[0, 0] = 0.0

    out_ref[0, 0] += s / n_tokens


def kernel(z, codebook):
    n, d = z.shape
    k = codebook.shape[0]
    zt = z.T
    out = pl.pallas_call(
        functools.partial(_vq_loss_kernel, n_tokens=n, k_codes=k),
        grid=(n // _BM,),
        in_specs=[
            pl.BlockSpec((d, _BM), lambda m: (0, m)),
            pl.BlockSpec((k, d), lambda m: (0, 0)),
        ],
        out_specs=pl.BlockSpec(memory_space=pltpu.SMEM),
        out_shape=jax.ShapeDtypeStruct((1, 1), jnp.float32),
        scratch_shapes=[pltpu.VMEM((k, d + 1), jnp.bfloat16)],
    )(zt, codebook)
    return out[0, 0]


# speedup vs baseline: 1.9133x; 1.1228x over previous
import functools

import jax
import jax.numpy as jnp
from jax.experimental import pallas as pl
from jax.experimental.pallas import tpu as pltpu

_BM = 256
_BK = 1024


def _vq_loss_kernel(zt_ref, c_ref, out_ref, c2_ref, *, n_tokens, k_codes):
    d = zt_ref.shape[0]

    @pl.when(pl.program_id(0) == 0)
    def _augment():
        c = c_ref[:]
        c2_ref[:, 0:d] = c.astype(jnp.bfloat16)
        c2_ref[:, d:d + 1] = jnp.sum(c * c, axis=1, keepdims=True
                                     ).astype(jnp.bfloat16)

    ztb = zt_ref[:]
    z2 = jnp.concatenate(
        [-2.0 * ztb, jnp.ones((1, ztb.shape[1]), jnp.float32)],
        axis=0).astype(jnp.bfloat16)

    def body(i, minv):
        c2 = c2_ref[pl.ds(i * _BK, _BK), :]
        part = jnp.dot(c2, z2, preferred_element_type=jnp.float32)
        return jnp.minimum(minv, jnp.min(part, axis=0, keepdims=True))

    minv = jax.lax.fori_loop(
        0, k_codes // _BK, body,
        jnp.full((1, ztb.shape[1]), jnp.inf, dtype=jnp.float32),
        unroll=8)
    zsq = jnp.sum(ztb * ztb, axis=0, keepdims=True)
    s = jnp.sum(minv + zsq)

    @pl.when(pl.program_id(0) == 0)
    def _init():
        out_ref[0, 0] = 0.0

    out_ref[0, 0] += s / n_tokens


def kernel(z, codebook):
    n, d = z.shape
    k = codebook.shape[0]
    zt = z.T
    out = pl.pallas_call(
        functools.partial(_vq_loss_kernel, n_tokens=n, k_codes=k),
        grid=(n // _BM,),
        in_specs=[
            pl.BlockSpec((d, _BM), lambda m: (0, m)),
            pl.BlockSpec((k, d), lambda m: (0, 0)),
        ],
        out_specs=pl.BlockSpec(memory_space=pltpu.SMEM),
        out_shape=jax.ShapeDtypeStruct((1, 1), jnp.float32),
        scratch_shapes=[pltpu.VMEM((k, d + 1), jnp.bfloat16)],
    )(zt, codebook)
    return out[0, 0]


# speedup vs baseline: 2.1585x; 1.1281x over previous
import functools

import jax
import jax.numpy as jnp
from jax.experimental import pallas as pl
from jax.experimental.pallas import tpu as pltpu

_BM = 512
_BK = 1024


def _vq_loss_kernel(zt_ref, c_ref, out_ref, c2_ref, *, n_tokens, k_codes):
    d = zt_ref.shape[0]

    @pl.when(pl.program_id(0) == 0)
    def _augment():
        c = c_ref[:]
        c2_ref[:, 0:d] = c.astype(jnp.bfloat16)
        c2_ref[:, d:d + 1] = jnp.sum(c * c, axis=1, keepdims=True
                                     ).astype(jnp.bfloat16)

    ztb = zt_ref[:]
    z2 = jnp.concatenate(
        [-2.0 * ztb, jnp.ones((1, ztb.shape[1]), jnp.float32)],
        axis=0).astype(jnp.bfloat16)

    def body(i, minv):
        c2 = c2_ref[pl.ds(i * _BK, _BK), :]
        part = jnp.dot(c2, z2, preferred_element_type=jnp.float32)
        return jnp.minimum(minv, jnp.min(part, axis=0, keepdims=True))

    minv = jax.lax.fori_loop(
        0, k_codes // _BK, body,
        jnp.full((1, ztb.shape[1]), jnp.inf, dtype=jnp.float32),
        unroll=8)
    zsq = jnp.sum(ztb * ztb, axis=0, keepdims=True)
    s = jnp.sum(minv + zsq)

    @pl.when(pl.program_id(0) == 0)
    def _init():
        out_ref[0, 0] = 0.0

    out_ref[0, 0] += s / n_tokens


def kernel(z, codebook):
    n, d = z.shape
    k = codebook.shape[0]
    zt = z.T
    out = pl.pallas_call(
        functools.partial(_vq_loss_kernel, n_tokens=n, k_codes=k),
        grid=(n // _BM,),
        in_specs=[
            pl.BlockSpec((d, _BM), lambda m: (0, m)),
            pl.BlockSpec((k, d), lambda m: (0, 0)),
        ],
        out_specs=pl.BlockSpec(memory_space=pltpu.SMEM),
        out_shape=jax.ShapeDtypeStruct((1, 1), jnp.float32),
        scratch_shapes=[pltpu.VMEM((k, d + 1), jnp.bfloat16)],
    )(zt, codebook)
    return out[0, 0]


# speedup vs baseline: 2.3041x; 1.0675x over previous
import functools

import jax
import jax.numpy as jnp
from jax.experimental import pallas as pl
from jax.experimental.pallas import tpu as pltpu

_BM = 1024
_BK = 1024


def _vq_loss_kernel(zt_ref, c_ref, out_ref, c2_ref, *, n_tokens, k_codes):
    d = zt_ref.shape[0]

    @pl.when(pl.program_id(0) == 0)
    def _augment():
        c = c_ref[:]
        c2_ref[:, 0:d] = c.astype(jnp.bfloat16)
        c2_ref[:, d:d + 1] = jnp.sum(c * c, axis=1, keepdims=True
                                     ).astype(jnp.bfloat16)

    ztb = zt_ref[:]
    z2 = jnp.concatenate(
        [-2.0 * ztb, jnp.ones((1, ztb.shape[1]), jnp.float32)],
        axis=0).astype(jnp.bfloat16)

    def body(i, minv):
        c2 = c2_ref[pl.ds(i * _BK, _BK), :]
        part = jnp.dot(c2, z2, preferred_element_type=jnp.float32)
        return jnp.minimum(minv, jnp.min(part, axis=0, keepdims=True))

    minv = jax.lax.fori_loop(
        0, k_codes // _BK, body,
        jnp.full((1, ztb.shape[1]), jnp.inf, dtype=jnp.float32),
        unroll=8)
    zsq = jnp.sum(ztb * ztb, axis=0, keepdims=True)
    s = jnp.sum(minv + zsq)

    @pl.when(pl.program_id(0) == 0)
    def _init():
        out_ref[0, 0] = 0.0

    out_ref[0, 0] += s / n_tokens


def kernel(z, codebook):
    n, d = z.shape
    k = codebook.shape[0]
    zt = z.T
    out = pl.pallas_call(
        functools.partial(_vq_loss_kernel, n_tokens=n, k_codes=k),
        grid=(n // _BM,),
        in_specs=[
            pl.BlockSpec((d, _BM), lambda m: (0, m)),
            pl.BlockSpec((k, d), lambda m: (0, 0)),
        ],
        out_specs=pl.BlockSpec(memory_space=pltpu.SMEM),
        out_shape=jax.ShapeDtypeStruct((1, 1), jnp.float32),
        scratch_shapes=[pltpu.VMEM((k, d + 1), jnp.bfloat16)],
    )(zt, codebook)
    return out[0, 0]


# speedup vs baseline: 2.3875x; 1.0362x over previous
import functools

import jax
import jax.numpy as jnp
from jax.experimental import pallas as pl
from jax.experimental.pallas import tpu as pltpu

_BM = 2048
_BK = 1024


def _vq_loss_kernel(zt_ref, c_ref, out_ref, c2_ref, *, n_tokens, k_codes):
    d = zt_ref.shape[0]

    @pl.when(pl.program_id(0) == 0)
    def _augment():
        c = c_ref[:]
        c2_ref[:, 0:d] = c.astype(jnp.bfloat16)
        c2_ref[:, d:d + 1] = jnp.sum(c * c, axis=1, keepdims=True
                                     ).astype(jnp.bfloat16)

    ztb = zt_ref[:]
    z2 = jnp.concatenate(
        [-2.0 * ztb, jnp.ones((1, ztb.shape[1]), jnp.float32)],
        axis=0).astype(jnp.bfloat16)

    def body(i, minv):
        c2 = c2_ref[pl.ds(i * _BK, _BK), :]
        part = jnp.dot(c2, z2, preferred_element_type=jnp.float32)
        return jnp.minimum(minv, jnp.min(part, axis=0, keepdims=True))

    minv = jax.lax.fori_loop(
        0, k_codes // _BK, body,
        jnp.full((1, ztb.shape[1]), jnp.inf, dtype=jnp.float32),
        unroll=8)
    zsq = jnp.sum(ztb * ztb, axis=0, keepdims=True)
    s = jnp.sum(minv + zsq)

    @pl.when(pl.program_id(0) == 0)
    def _init():
        out_ref[0, 0] = 0.0

    out_ref[0, 0] += s / n_tokens


def kernel(z, codebook):
    n, d = z.shape
    k = codebook.shape[0]
    zt = z.T
    out = pl.pallas_call(
        functools.partial(_vq_loss_kernel, n_tokens=n, k_codes=k),
        grid=(n // _BM,),
        in_specs=[
            pl.BlockSpec((d, _BM), lambda m: (0, m)),
            pl.BlockSpec((k, d), lambda m: (0, 0)),
        ],
        out_specs=pl.BlockSpec(memory_space=pltpu.SMEM),
        out_shape=jax.ShapeDtypeStruct((1, 1), jnp.float32),
        scratch_shapes=[pltpu.VMEM((k, d + 1), jnp.bfloat16)],
    )(zt, codebook)
    return out[0, 0]


# speedup vs baseline: 2.4283x; 1.0171x over previous
import functools

import jax
import jax.numpy as jnp
from jax.experimental import pallas as pl
from jax.experimental.pallas import tpu as pltpu

_BM = 4096
_BK = 1024


def _vq_loss_kernel(zt_ref, c_ref, out_ref, c2_ref, *, n_tokens, k_codes):
    d = zt_ref.shape[0]

    @pl.when(pl.program_id(0) == 0)
    def _augment():
        c = c_ref[:]
        c2_ref[:, 0:d] = c.astype(jnp.bfloat16)
        c2_ref[:, d:d + 1] = jnp.sum(c * c, axis=1, keepdims=True
                                     ).astype(jnp.bfloat16)

    ztb = zt_ref[:]
    z2 = jnp.concatenate(
        [-2.0 * ztb, jnp.ones((1, ztb.shape[1]), jnp.float32)],
        axis=0).astype(jnp.bfloat16)

    def body(i, minv):
        c2 = c2_ref[pl.ds(i * _BK, _BK), :]
        part = jnp.dot(c2, z2, preferred_element_type=jnp.float32)
        return jnp.minimum(minv, jnp.min(part, axis=0, keepdims=True))

    minv = jax.lax.fori_loop(
        0, k_codes // _BK, body,
        jnp.full((1, ztb.shape[1]), jnp.inf, dtype=jnp.float32),
        unroll=8)
    zsq = jnp.sum(ztb * ztb, axis=0, keepdims=True)
    s = jnp.sum(minv + zsq)

    @pl.when(pl.program_id(0) == 0)
    def _init():
        out_ref[0, 0] = 0.0

    out_ref[0, 0] += s / n_tokens


def kernel(z, codebook):
    n, d = z.shape
    k = codebook.shape[0]
    zt = z.T
    out = pl.pallas_call(
        functools.partial(_vq_loss_kernel, n_tokens=n, k_codes=k),
        grid=(n // _BM,),
        in_specs=[
            pl.BlockSpec((d, _BM), lambda m: (0, m)),
            pl.BlockSpec((k, d), lambda m: (0, 0)),
        ],
        out_specs=pl.BlockSpec(memory_space=pltpu.SMEM),
        out_shape=jax.ShapeDtypeStruct((1, 1), jnp.float32),
        scratch_shapes=[pltpu.VMEM((k, d + 1), jnp.bfloat16)],
    )(zt, codebook)
    return out[0, 0]


# speedup vs baseline: 2.4328x; 1.0019x over previous
import functools

import jax
import jax.numpy as jnp
from jax.experimental import pallas as pl
from jax.experimental.pallas import tpu as pltpu

_BM = 4608
_BK = 1024


def _vq_loss_kernel(zt_ref, c_ref, out_ref, c2_ref, *, n_tokens, k_codes):
    d = zt_ref.shape[0]

    @pl.when(pl.program_id(0) == 0)
    def _augment():
        c = c_ref[:]
        c2_ref[:, 0:d] = c.astype(jnp.bfloat16)
        c2_ref[:, d:d + 1] = jnp.sum(c * c, axis=1, keepdims=True
                                     ).astype(jnp.bfloat16)

    ztb = zt_ref[:]
    z2 = jnp.concatenate(
        [-2.0 * ztb, jnp.ones((1, ztb.shape[1]), jnp.float32)],
        axis=0).astype(jnp.bfloat16)

    def body(i, minv):
        c2 = c2_ref[pl.ds(i * _BK, _BK), :]
        part = jnp.dot(c2, z2, preferred_element_type=jnp.float32)
        return jnp.minimum(minv, jnp.min(part, axis=0, keepdims=True))

    minv = jax.lax.fori_loop(
        0, k_codes // _BK, body,
        jnp.full((1, ztb.shape[1]), jnp.inf, dtype=jnp.float32),
        unroll=8)
    zsq = jnp.sum(ztb * ztb, axis=0, keepdims=True)
    s = jnp.sum(minv + zsq)

    @pl.when(pl.program_id(0) == 0)
    def _init():
        out_ref[0, 0] = 0.0

    out_ref[0, 0] += s / n_tokens


def kernel(z, codebook):
    n, d = z.shape
    k = codebook.shape[0]
    zt = z.T
    out = pl.pallas_call(
        functools.partial(_vq_loss_kernel, n_tokens=n, k_codes=k),
        grid=(n // _BM,),
        in_specs=[
            pl.BlockSpec((d, _BM), lambda m: (0, m)),
            pl.BlockSpec((k, d), lambda m: (0, 0)),
        ],
        out_specs=pl.BlockSpec(memory_space=pltpu.SMEM),
        out_shape=jax.ShapeDtypeStruct((1, 1), jnp.float32),
        scratch_shapes=[pltpu.VMEM((k, d + 1), jnp.bfloat16)],
    )(zt, codebook)
    return out[0, 0]


# speedup vs baseline: 2.4332x; 1.0001x over previous
import functools

import jax
import jax.numpy as jnp
from jax.experimental import pallas as pl
from jax.experimental.pallas import tpu as pltpu

_BM = 4608
_BK = 2048


def _vq_loss_kernel(zt_ref, c_ref, out_ref, c2_ref, *, n_tokens, k_codes):
    d = zt_ref.shape[0]

    @pl.when(pl.program_id(0) == 0)
    def _augment():
        c = c_ref[:]
        c2_ref[:, 0:d] = c.astype(jnp.bfloat16)
        c2_ref[:, d:d + 1] = jnp.sum(c * c, axis=1, keepdims=True
                                     ).astype(jnp.bfloat16)

    ztb = zt_ref[:]
    z2 = jnp.concatenate(
        [-2.0 * ztb, jnp.ones((1, ztb.shape[1]), jnp.float32)],
        axis=0).astype(jnp.bfloat16)

    def body(i, minv):
        c2 = c2_ref[pl.ds(i * _BK, _BK), :]
        part = jnp.dot(c2, z2, preferred_element_type=jnp.float32)
        return jnp.minimum(minv, jnp.min(part, axis=0, keepdims=True))

    minv = jax.lax.fori_loop(
        0, k_codes // _BK, body,
        jnp.full((1, ztb.shape[1]), jnp.inf, dtype=jnp.float32),
        unroll=4)
    zsq = jnp.sum(ztb * ztb, axis=0, keepdims=True)
    s = jnp.sum(minv + zsq)

    @pl.when(pl.program_id(0) == 0)
    def _init():
        out_ref[0, 0] = 0.0

    out_ref[0, 0] += s / n_tokens


def kernel(z, codebook):
    n, d = z.shape
    k = codebook.shape[0]
    zt = z.T
    out = pl.pallas_call(
        functools.partial(_vq_loss_kernel, n_tokens=n, k_codes=k),
        grid=(n // _BM,),
        in_specs=[
            pl.BlockSpec((d, _BM), lambda m: (0, m)),
            pl.BlockSpec((k, d), lambda m: (0, 0)),
        ],
        out_specs=pl.BlockSpec(memory_space=pltpu.SMEM),
        out_shape=jax.ShapeDtypeStruct((1, 1), jnp.float32),
        scratch_shapes=[pltpu.VMEM((k, d + 1), jnp.bfloat16)],
    )(zt, codebook)
    return out[0, 0]


# speedup vs baseline: 2.4535x; 1.0084x over previous
import functools

import jax
import jax.numpy as jnp
from jax.experimental import pallas as pl
from jax.experimental.pallas import tpu as pltpu

_BM = 4608
_BK = 2048


def _vq_loss_kernel(zt_ref, c_ref, out_ref, c2_ref, *, n_tokens, k_codes):
    d = zt_ref.shape[0]

    @pl.when(pl.program_id(0) == 0)
    def _augment():
        c = c_ref[:]
        c2_ref[:, 0:d] = c
        c2_ref[:, d:d + 1] = jnp.sum(c * c, axis=1, keepdims=True
                                     )

    ztb = zt_ref[:]
    z2 = jnp.concatenate(
        [-2.0 * ztb, jnp.ones((1, ztb.shape[1]), jnp.float32)],
        axis=0)

    def body(i, minv):
        c2 = c2_ref[pl.ds(i * _BK, _BK), :]
        part = jnp.dot(c2, z2, preferred_element_type=jnp.float32)
        return jnp.minimum(minv, jnp.min(part, axis=0, keepdims=True))

    minv = jax.lax.fori_loop(
        0, k_codes // _BK, body,
        jnp.full((1, ztb.shape[1]), jnp.inf, dtype=jnp.float32),
        unroll=4)
    zsq = jnp.sum(ztb * ztb, axis=0, keepdims=True)
    s = jnp.sum(minv + zsq)

    @pl.when(pl.program_id(0) == 0)
    def _init():
        out_ref[0, 0] = 0.0

    out_ref[0, 0] += s / n_tokens


def kernel(z, codebook):
    n, d = z.shape
    k = codebook.shape[0]
    zt = z.T
    out = pl.pallas_call(
        functools.partial(_vq_loss_kernel, n_tokens=n, k_codes=k),
        grid=(n // _BM,),
        in_specs=[
            pl.BlockSpec((d, _BM), lambda m: (0, m)),
            pl.BlockSpec((k, d), lambda m: (0, 0)),
        ],
        out_specs=pl.BlockSpec(memory_space=pltpu.SMEM),
        out_shape=jax.ShapeDtypeStruct((1, 1), jnp.float32),
        scratch_shapes=[pltpu.VMEM((k, d + 1), jnp.float32)],
    )(zt, codebook)
    return out[0, 0]
